# Initial kernel scaffold; baseline (speedup 1.0000x reference)
import jax
import jax.numpy as jnp
from jax.experimental import pallas as pl


def kernel(node_attrs, node_feats, edge_attrs, edge_feats, edge_index, W_up, W_down, W_skip, W1, W2, W3, W4, W_out):
    raise NotImplementedError("write your pallas kernel here")


# speedup vs baseline: 2.4463x; 2.4463x over previous
import functools

import jax
import jax.numpy as jnp
from jax import lax
from jax.experimental import pallas as pl
from jax.experimental.pallas import tpu as pltpu
from jax.experimental.pallas import tpu_sc as plsc

N_NODES = 10000
N_PAD = 10240
E_EDGES = 320000
NC, NS = 2, 16
NW = NC * NS
EPW = E_EDGES // NW
CH = 80
NCHUNK = EPW // CH
ROWS_PER_TILE = N_PAD // NS
AVG = 32.0

def _sc_mesh():
    return plsc.VectorSubcoreMesh(
        core_axis_name="c", subcore_axis_name="s", num_cores=NC, num_subcores=NS)


def _node_mm_body(nf_ref, wup_ref, wdown_ref, wskip_ref,
                  up_ref, down_ref, sc_ref):
    nf = nf_ref[...]
    up_ref[...] = nf @ wup_ref[...]
    down_ref[...] = nf @ wdown_ref[...]
    sc_ref[...] = nf @ wskip_ref[...]


def _node_matmuls(node_feats, w_up, w_down, w_skip):
    n = node_feats.shape[0]
    return pl.pallas_call(
        _node_mm_body,
        out_shape=[
            jax.ShapeDtypeStruct((n, 128), jnp.float32),
            jax.ShapeDtypeStruct((n, 64), jnp.float32),
            jax.ShapeDtypeStruct((n, 128), jnp.float32),
        ],
    )(node_feats, w_up, w_down, w_skip)


def _gather_down_body(down_hbm, send_hbm, recv_hbm, outs_hbm, outr_hbm,
                      sidx, ridx, srows, rrows, sem_s, sem_r):
    wid = lax.axis_index("s") * NC + lax.axis_index("c")
    base0 = wid * EPW

    def step(i, carry):
        base = base0 + i * CH
        pltpu.sync_copy(send_hbm.at[pl.ds(base, CH)], sidx)
        pltpu.sync_copy(recv_hbm.at[pl.ds(base, CH)], ridx)
        cs = pltpu.async_copy(down_hbm.at[sidx], srows, sem_s)
        cr = pltpu.async_copy(down_hbm.at[ridx], rrows, sem_r)
        cs.wait()
        cr.wait()
        pltpu.sync_copy(srows, outs_hbm.at[pl.ds(base, CH)])
        pltpu.sync_copy(rrows, outr_hbm.at[pl.ds(base, CH)])
        return carry

    lax.fori_loop(0, NCHUNK, step, 0)


def _gather_down(down, sender, receiver):
    f = functools.partial(
        pl.kernel,
        out_type=[
            jax.ShapeDtypeStruct((E_EDGES, 64), jnp.float32),
            jax.ShapeDtypeStruct((E_EDGES, 64), jnp.float32),
        ],
        mesh=_sc_mesh(),
        scratch_types=[
            pltpu.VMEM((CH,), jnp.int32),
            pltpu.VMEM((CH,), jnp.int32),
            pltpu.VMEM((CH, 64), jnp.float32),
            pltpu.VMEM((CH, 64), jnp.float32),
            pltpu.SemaphoreType.DMA,
            pltpu.SemaphoreType.DMA,
        ],
        compiler_params=pltpu.CompilerParams(use_tc_tiling_on_sc=False),
    )(_gather_down_body)
    return f(down, sender, receiver)


BE = 2000


def _silu(x):
    return x * jax.nn.sigmoid(x)


def _mlp_body(ef_ref, ds_ref, dr_ref, ea_ref, w1_ref, w2_ref, w3_ref, w4_ref,
              out_ref):
    x = jnp.concatenate([ef_ref[...], ds_ref[...], dr_ref[...]], axis=-1)
    h = _silu(x @ w1_ref[...])
    h = _silu(h @ w2_ref[...])
    h = _silu(h @ w3_ref[...])
    out_ref[...] = (h @ w4_ref[...]) * ea_ref[...]


def _edge_mlp(edge_feats, downs, downr, edge_attrs, w1, w2, w3, w4):
    grid = (E_EDGES // BE,)
    return pl.pallas_call(
        _mlp_body,
        grid=grid,
        in_specs=[
            pl.BlockSpec((BE, 8), lambda i: (i, 0)),
            pl.BlockSpec((BE, 64), lambda i: (i, 0)),
            pl.BlockSpec((BE, 64), lambda i: (i, 0)),
            pl.BlockSpec((BE, 1), lambda i: (i, 0)),
            pl.BlockSpec((136, 256), lambda i: (0, 0)),
            pl.BlockSpec((256, 256), lambda i: (0, 0)),
            pl.BlockSpec((256, 256), lambda i: (0, 0)),
            pl.BlockSpec((256, 128), lambda i: (0, 0)),
        ],
        out_specs=pl.BlockSpec((BE, 128), lambda i: (i, 0)),
        out_shape=jax.ShapeDtypeStruct((E_EDGES, 128), jnp.float32),
    )(edge_feats, downs, downr, edge_attrs, w1, w2, w3, w4)


def _scatter_body(up_hbm, send_hbm, recv_hbm, w_hbm, out_hbm,
                  sidx, ridx, rows, wrows, zbuf, acc, sem):
    c = lax.axis_index("c")
    s = lax.axis_index("s")
    wid = s * NC + c

    def zrow(j, carry):
        for k in range(8):
            zbuf[j, pl.ds(k * 16, 16)] = jnp.zeros((16,), jnp.float32)
        return carry

    lax.fori_loop(0, 128, zrow, 0)
    for t in range(ROWS_PER_TILE // 128):
        pltpu.sync_copy(zbuf, acc.at[pl.ds(s * ROWS_PER_TILE + t * 128, 128)])
    plsc.subcore_barrier()

    base0 = wid * EPW

    def step(i, carry):
        base = base0 + i * CH
        pltpu.sync_copy(send_hbm.at[pl.ds(base, CH)], sidx)
        pltpu.sync_copy(recv_hbm.at[pl.ds(base, CH)], ridx)
        cp = pltpu.async_copy(up_hbm.at[sidx], rows, sem)
        pltpu.sync_copy(w_hbm.at[pl.ds(base, CH)], wrows)
        cp.wait()

        def mul(j, carry2):
            for k in range(8):
                sl = pl.ds(k * 16, 16)
                rows[j, sl] = rows[j, sl] * wrows[j, sl]
            return carry2

        lax.fori_loop(0, CH, mul, 0)
        pltpu.sync_copy(rows, acc.at[ridx], add=True)
        return carry

    lax.fori_loop(0, NCHUNK, step, 0)
    plsc.subcore_barrier()
    pltpu.sync_copy(acc.at[pl.ds(s * ROWS_PER_TILE, ROWS_PER_TILE)],
                    out_hbm.at[c, pl.ds(s * ROWS_PER_TILE, ROWS_PER_TILE)])


def _scatter_up(up, sender, receiver, w):
    f = functools.partial(
        pl.kernel,
        out_type=jax.ShapeDtypeStruct((NC, N_PAD, 128), jnp.float32),
        mesh=_sc_mesh(),
        scratch_types=[
            pltpu.VMEM((CH,), jnp.int32),
            pltpu.VMEM((CH,), jnp.int32),
            pltpu.VMEM((CH, 128), jnp.float32),
            pltpu.VMEM((CH, 128), jnp.float32),
            pltpu.VMEM((128, 128), jnp.float32),
            pltpu.VMEM_SHARED((N_PAD, 128), jnp.float32),
            pltpu.SemaphoreType.DMA,
        ],
        compiler_params=pltpu.CompilerParams(use_tc_tiling_on_sc=False),
    )(_scatter_body)
    return f(up, sender, receiver, w)


def _out_mm_body(p0_ref, p1_ref, wout_ref, out_ref):
    msg = p0_ref[...] + p1_ref[...]
    out_ref[...] = (msg @ wout_ref[...]) * (1.0 / AVG)


def _out_matmul(p0, p1, w_out):
    n = p0.shape[0]
    return pl.pallas_call(
        _out_mm_body,
        out_shape=jax.ShapeDtypeStruct((n, 128), jnp.float32),
    )(p0, p1, w_out)


def kernel(node_attrs, node_feats, edge_attrs, edge_feats, edge_index,
           W_up, W_down, W_skip, W1, W2, W3, W4, W_out):
    del node_attrs
    sender = edge_index[0]
    receiver = edge_index[1]

    up, down, sc = _node_matmuls(node_feats, W_up, W_down, W_skip)
    downs, downr = _gather_down(down, sender, receiver)
    w = _edge_mlp(edge_feats, downs, downr, edge_attrs, W1, W2, W3, W4)
    partials = _scatter_up(up, sender, receiver, w)
    message = _out_matmul(partials[0, :N_NODES], partials[1, :N_NODES], W_out)
    return (message.reshape(N_NODES, 128, 1), sc)


# speedup vs baseline: 2.4477x; 1.0006x over previous
import functools

import jax
import jax.numpy as jnp
from jax import lax
from jax.experimental import pallas as pl
from jax.experimental.pallas import tpu as pltpu
from jax.experimental.pallas import tpu_sc as plsc

N_NODES = 10000
N_PAD = 10240
E_EDGES = 320000
NC, NS = 2, 16
NW = NC * NS
EPW = E_EDGES // NW
CH = 80
NCHUNK = EPW // CH
ROWS_PER_TILE = N_PAD // NS
AVG = 32.0

def _sc_mesh():
    return plsc.VectorSubcoreMesh(
        core_axis_name="c", subcore_axis_name="s", num_cores=NC, num_subcores=NS)


def _node_mm_body(nf_ref, wup_ref, wdown_ref, wskip_ref,
                  up_ref, down_ref, sc_ref):
    nf = nf_ref[...]
    up_ref[...] = nf @ wup_ref[...]
    down_ref[...] = nf @ wdown_ref[...]
    sc_ref[...] = nf @ wskip_ref[...]


def _node_matmuls(node_feats, w_up, w_down, w_skip):
    n = node_feats.shape[0]
    return pl.pallas_call(
        _node_mm_body,
        out_shape=[
            jax.ShapeDtypeStruct((n, 128), jnp.float32),
            jax.ShapeDtypeStruct((n, 64), jnp.float32),
            jax.ShapeDtypeStruct((n, 128), jnp.float32),
        ],
    )(node_feats, w_up, w_down, w_skip)


def _gather_down_body(down_hbm, send_hbm, recv_hbm, outs_hbm, outr_hbm,
                      sidx, ridx, srows, rrows, sem_s, sem_r):
    wid = lax.axis_index("s") * NC + lax.axis_index("c")
    base0 = wid * EPW

    def step(i, carry):
        base = base0 + i * CH
        pltpu.sync_copy(send_hbm.at[pl.ds(base, CH)], sidx)
        pltpu.sync_copy(recv_hbm.at[pl.ds(base, CH)], ridx)
        cs = pltpu.async_copy(down_hbm.at[sidx], srows, sem_s)
        cr = pltpu.async_copy(down_hbm.at[ridx], rrows, sem_r)
        cs.wait()
        cr.wait()
        pltpu.sync_copy(srows, outs_hbm.at[pl.ds(base, CH)])
        pltpu.sync_copy(rrows, outr_hbm.at[pl.ds(base, CH)])
        return carry

    lax.fori_loop(0, NCHUNK, step, 0)


def _gather_down(down, sender, receiver):
    f = functools.partial(
        pl.kernel,
        out_type=[
            jax.ShapeDtypeStruct((E_EDGES, 64), jnp.float32),
            jax.ShapeDtypeStruct((E_EDGES, 64), jnp.float32),
        ],
        mesh=_sc_mesh(),
        scratch_types=[
            pltpu.VMEM((CH,), jnp.int32),
            pltpu.VMEM((CH,), jnp.int32),
            pltpu.VMEM((CH, 64), jnp.float32),
            pltpu.VMEM((CH, 64), jnp.float32),
            pltpu.SemaphoreType.DMA,
            pltpu.SemaphoreType.DMA,
        ],
        compiler_params=pltpu.CompilerParams(use_tc_tiling_on_sc=False),
    )(_gather_down_body)
    return f(down, sender, receiver)


BE = 2000


def _silu(x):
    return x * jax.nn.sigmoid(x)


def _mlp_body(ef_ref, ds_ref, dr_ref, ea_ref, w1_ref, w2_ref, w3_ref, w4_ref,
              out_ref):
    f32 = jnp.float32
    bf = jnp.bfloat16
    x = jnp.concatenate([ef_ref[...], ds_ref[...], dr_ref[...]], axis=-1)
    h = _silu(jnp.dot(x.astype(bf), w1_ref[...], preferred_element_type=f32))
    h = _silu(jnp.dot(h.astype(bf), w2_ref[...], preferred_element_type=f32))
    h = _silu(jnp.dot(h.astype(bf), w3_ref[...], preferred_element_type=f32))
    out_ref[...] = jnp.dot(h.astype(bf), w4_ref[...],
                           preferred_element_type=f32) * ea_ref[...]


def _edge_mlp(edge_feats, downs, downr, edge_attrs, w1, w2, w3, w4):
    grid = (E_EDGES // BE,)
    return pl.pallas_call(
        _mlp_body,
        grid=grid,
        in_specs=[
            pl.BlockSpec((BE, 8), lambda i: (i, 0)),
            pl.BlockSpec((BE, 64), lambda i: (i, 0)),
            pl.BlockSpec((BE, 64), lambda i: (i, 0)),
            pl.BlockSpec((BE, 1), lambda i: (i, 0)),
            pl.BlockSpec((136, 256), lambda i: (0, 0)),
            pl.BlockSpec((256, 256), lambda i: (0, 0)),
            pl.BlockSpec((256, 256), lambda i: (0, 0)),
            pl.BlockSpec((256, 128), lambda i: (0, 0)),
        ],
        out_specs=pl.BlockSpec((BE, 128), lambda i: (i, 0)),
        out_shape=jax.ShapeDtypeStruct((E_EDGES, 128), jnp.float32),
    )(edge_feats, downs, downr, edge_attrs,
      w1.astype(jnp.bfloat16), w2.astype(jnp.bfloat16),
      w3.astype(jnp.bfloat16), w4.astype(jnp.bfloat16))


def _scatter_body(up_hbm, send_hbm, recv_hbm, w_hbm, out_hbm,
                  sidx, ridx, rows, wrows, zbuf, acc, sem):
    c = lax.axis_index("c")
    s = lax.axis_index("s")
    wid = s * NC + c

    def zrow(j, carry):
        for k in range(8):
            zbuf[j, pl.ds(k * 16, 16)] = jnp.zeros((16,), jnp.float32)
        return carry

    lax.fori_loop(0, 128, zrow, 0)
    for t in range(ROWS_PER_TILE // 128):
        pltpu.sync_copy(zbuf, acc.at[pl.ds(s * ROWS_PER_TILE + t * 128, 128)])
    plsc.subcore_barrier()

    base0 = wid * EPW

    def step(i, carry):
        base = base0 + i * CH
        pltpu.sync_copy(send_hbm.at[pl.ds(base, CH)], sidx)
        pltpu.sync_copy(recv_hbm.at[pl.ds(base, CH)], ridx)
        cp = pltpu.async_copy(up_hbm.at[sidx], rows, sem)
        pltpu.sync_copy(w_hbm.at[pl.ds(base, CH)], wrows)
        cp.wait()

        def mul(j, carry2):
            for k in range(8):
                sl = pl.ds(k * 16, 16)
                rows[j, sl] = rows[j, sl] * wrows[j, sl]
            return carry2

        lax.fori_loop(0, CH, mul, 0)
        pltpu.sync_copy(rows, acc.at[ridx], add=True)
        return carry

    lax.fori_loop(0, NCHUNK, step, 0)
    plsc.subcore_barrier()
    pltpu.sync_copy(acc.at[pl.ds(s * ROWS_PER_TILE, ROWS_PER_TILE)],
                    out_hbm.at[c, pl.ds(s * ROWS_PER_TILE, ROWS_PER_TILE)])


def _scatter_up(up, sender, receiver, w):
    f = functools.partial(
        pl.kernel,
        out_type=jax.ShapeDtypeStruct((NC, N_PAD, 128), jnp.float32),
        mesh=_sc_mesh(),
        scratch_types=[
            pltpu.VMEM((CH,), jnp.int32),
            pltpu.VMEM((CH,), jnp.int32),
            pltpu.VMEM((CH, 128), jnp.float32),
            pltpu.VMEM((CH, 128), jnp.float32),
            pltpu.VMEM((128, 128), jnp.float32),
            pltpu.VMEM_SHARED((N_PAD, 128), jnp.float32),
            pltpu.SemaphoreType.DMA,
        ],
        compiler_params=pltpu.CompilerParams(use_tc_tiling_on_sc=False),
    )(_scatter_body)
    return f(up, sender, receiver, w)


def _out_mm_body(p0_ref, p1_ref, wout_ref, out_ref):
    msg = p0_ref[...] + p1_ref[...]
    out_ref[...] = (msg @ wout_ref[...]) * (1.0 / AVG)


def _out_matmul(p0, p1, w_out):
    n = p0.shape[0]
    return pl.pallas_call(
        _out_mm_body,
        out_shape=jax.ShapeDtypeStruct((n, 128), jnp.float32),
    )(p0, p1, w_out)


def kernel(node_attrs, node_feats, edge_attrs, edge_feats, edge_index,
           W_up, W_down, W_skip, W1, W2, W3, W4, W_out):
    del node_attrs
    sender = edge_index[0]
    receiver = edge_index[1]

    up, down, sc = _node_matmuls(node_feats, W_up, W_down, W_skip)
    downs, downr = _gather_down(down, sender, receiver)
    w = _edge_mlp(edge_feats, downs, downr, edge_attrs, W1, W2, W3, W4)
    partials = _scatter_up(up, sender, receiver, w)
    message = _out_matmul(partials[0, :N_NODES], partials[1, :N_NODES], W_out)
    return (message.reshape(N_NODES, 128, 1), sc)
